# baseline (speedup 1.0000x reference)
import functools

import jax
import jax.numpy as jnp
from jax import lax
from jax.experimental import pallas as pl
from jax.experimental.pallas import tpu as pltpu
from jax.experimental.pallas import tpu_sc as plsc

_NUM_CODE = 1024
_CODE_DIM = 64
_COMMIT = 0.25
_TILE = 1152

_NC = 2
_NS = 16
_NW = _NC * _NS
_ROWS = 73728
_RPW = _ROWS // _NW
_CHUNK = 128
_ROWS_H = _ROWS // 2
_RPW_H = _ROWS_H // _NW
_NCHUNK_H = _RPW_H // _CHUNK
_GROUP = 3
_NGROUP = _NCHUNK_H // _GROUP
_PAD = 128


def _cb_prep_body(cb_ref, cbn_ref, cbnb_ref):
    cb = cb_ref[...]
    cbn = cb / jnp.maximum(
        jnp.sqrt(jnp.sum(cb * cb, axis=1, keepdims=True)), 1e-12)
    cbn_ref[...] = jnp.concatenate(
        [cbn, jnp.zeros((_NUM_CODE, _PAD - _CODE_DIM), jnp.float32)], axis=1)
    cbnb_ref[...] = cbn.astype(jnp.bfloat16)


def _cb_prep(codebook):
    return pl.pallas_call(
        _cb_prep_body,
        out_shape=[
            jax.ShapeDtypeStruct((_NUM_CODE, _PAD), jnp.float32),
            jax.ShapeDtypeStruct((_NUM_CODE, _CODE_DIM), jnp.bfloat16),
        ],
    )(codebook)


def _vq_tc_body(x_ref, cbnb_ref, idx_ref, maxsum_ref, q_ref):
    pid = pl.program_id(0)
    x = jnp.reshape(x_ref[...], (_TILE, _CODE_DIM))
    xn = x / jnp.maximum(
        jnp.sqrt(jnp.sum(x * x, axis=1, keepdims=True)), 1e-12)
    simT = lax.dot_general(
        cbnb_ref[...], xn.astype(jnp.bfloat16),
        (((1,), (1,)), ((), ())),
        preferred_element_type=jnp.float32)
    maxv = jnp.max(simT, axis=0)
    onehot = jnp.where(simT == maxv[None, :],
                       jnp.float32(1.0), jnp.float32(0.0))
    i2 = lax.broadcasted_iota(jnp.int32, (2, _NUM_CODE), 1)
    r2 = lax.broadcasted_iota(jnp.int32, (2, _NUM_CODE), 0)
    wt = jnp.where(r2 == 0, i2 & 255, i2 >> 8).astype(jnp.float32)
    res = lax.dot_general(wt, onehot, (((1,), (0,)), ((), ())),
                          preferred_element_type=jnp.float32)
    idx = (res[0] + jnp.float32(256.0) * res[1]).astype(jnp.int32)
    idx_ref[0, 0, :] = idx
    q = lax.dot_general(onehot.astype(jnp.bfloat16), cbnb_ref[...],
                        (((0,), (0,)), ((), ())),
                        preferred_element_type=jnp.float32)
    q_ref[...] = jnp.reshape(q, q_ref.shape)
    s = jnp.sum(maxv).reshape(1, 1)

    @pl.when(pid == 0)
    def _():
        maxsum_ref[...] = s

    @pl.when(pid != 0)
    def _():
        maxsum_ref[...] = maxsum_ref[...] + s


def _vq_tc(x4d, cbnb):
    b, c, t, d = x4d.shape
    grid = (b * c * t) // _TILE
    cpb = _TILE // t
    return pl.pallas_call(
        _vq_tc_body,
        grid=(grid,),
        in_specs=[
            pl.BlockSpec((1, cpb, t, d),
                         lambda i: (i // (8 // cpb), i % (8 // cpb), 0, 0)),
            pl.BlockSpec((_NUM_CODE, _CODE_DIM), lambda i: (0, 0)),
        ],
        out_specs=[
            pl.BlockSpec((1, 1, _TILE), lambda i: (i, 0, 0)),
            pl.BlockSpec((1, 1), lambda i: (0, 0)),
            pl.BlockSpec((1, cpb, t, d),
                         lambda i: (i // (8 // cpb), i % (8 // cpb), 0, 0)),
        ],
        out_shape=[
            jax.ShapeDtypeStruct((grid, 1, _TILE), jnp.int32),
            jax.ShapeDtypeStruct((1, 1), jnp.float32),
            jax.ShapeDtypeStruct((b, c, t, d), jnp.float32),
        ],
    )(x4d, cbnb)


def _vq_sc_body(idx_hbm, hist_hbm, idx_v, hist16_v, hist_v):
    wid = lax.axis_index("s") * _NC + lax.axis_index("c")

    pltpu.sync_copy(idx_hbm.at[wid], idx_v)

    zeros16 = jnp.zeros((_NS,), jnp.float32)

    def _zero(c, _):
        for l in range(_NS):
            hist16_v[l, pl.ds(c * _NS, _NS)] = zeros16
        return 0

    lax.fori_loop(0, _NUM_CODE // _NS, _zero, 0)

    lanes = lax.iota(jnp.int32, _NS)
    ones16 = jnp.ones((_NS,), jnp.float32)

    def _hist(g, _):
        codes = idx_v[pl.ds(g * _NS, _NS)]
        plsc.addupdate_scatter(hist16_v, [lanes, codes], ones16)
        return 0

    lax.fori_loop(0, _RPW // _NS, _hist, 0)

    def _red(c, _):
        acc = jnp.zeros((_NS,), jnp.float32)
        for l in range(_NS):
            acc = acc + hist16_v[l, pl.ds(c * _NS, _NS)]
        hist_v[0, pl.ds(c * _NS, _NS)] = acc
        return 0

    lax.fori_loop(0, _NUM_CODE // _NS, _red, 0)
    pltpu.sync_copy(hist_v, hist_hbm.at[wid].at[pl.ds(0, 1)])


def _vq_sc(idx2d):
    mesh = plsc.VectorSubcoreMesh(core_axis_name="c", subcore_axis_name="s")
    f = pl.kernel(
        _vq_sc_body,
        mesh=mesh,
        compiler_params=pltpu.CompilerParams(needs_layout_passes=False),
        out_type=jax.ShapeDtypeStruct((_NW, 8, _NUM_CODE), jnp.float32),
        scratch_types=[
            pltpu.VMEM((_RPW,), jnp.int32),
            pltpu.VMEM((_NS, _NUM_CODE), jnp.float32),
            pltpu.VMEM((1, _NUM_CODE), jnp.float32),
        ],
    )
    return f(idx2d)


def kernel(encoded_patch_input, codebook):
    b, c, t, d = encoded_patch_input.shape
    rows = b * c * t
    cbn, cbnb = _cb_prep(codebook)
    idx3, maxsum, quantized_st = _vq_tc(encoded_patch_input, cbnb)
    idx = idx3.reshape(rows)
    hist32 = _vq_sc(idx.reshape(_NW, _RPW))
    hist = jnp.sum(hist32[:, 0, :], axis=0)

    total = jnp.float32(rows * d)
    loss = (1.0 + _COMMIT) * (2.0 * rows - 2.0 * maxsum[0, 0]) / total
    avg_probs = hist / rows
    perplexity = jnp.exp(-jnp.sum(avg_probs * jnp.log(avg_probs + 1e-10)))
    code_indices = idx.reshape(b, c, t)
    return (loss, quantized_st, perplexity, codebook, code_indices)


# speedup vs baseline: 1.1696x; 1.1696x over previous
import functools

import jax
import jax.numpy as jnp
from jax import lax
from jax.experimental import pallas as pl
from jax.experimental.pallas import tpu as pltpu
from jax.experimental.pallas import tpu_sc as plsc

_NUM_CODE = 1024
_CODE_DIM = 64
_COMMIT = 0.25
_TILE = 1152

_NC = 2
_NS = 16
_NW = _NC * _NS
_ROWS = 73728
_RPW = _ROWS // _NW
_CHUNK = 128
_NCHUNK = _RPW // _CHUNK
_GROUP = 6
_PAD = 128


def _cb_prep_body(cb_ref, cbn_ref, cbnb_ref):
    cb = cb_ref[...]
    cbn = cb / jnp.maximum(
        jnp.sqrt(jnp.sum(cb * cb, axis=1, keepdims=True)), 1e-12)
    cbn_ref[...] = jnp.concatenate(
        [cbn, jnp.zeros((_NUM_CODE, _PAD - _CODE_DIM), jnp.float32)], axis=1)
    cbnb_ref[...] = cbn.astype(jnp.bfloat16)


def _cb_prep(codebook):
    return pl.pallas_call(
        _cb_prep_body,
        out_shape=[
            jax.ShapeDtypeStruct((_NUM_CODE, _PAD), jnp.float32),
            jax.ShapeDtypeStruct((_NUM_CODE, _CODE_DIM), jnp.bfloat16),
        ],
    )(codebook)


def _vq_tc_body(x_ref, cbnb_ref, idx_ref, maxsum_ref):
    pid = pl.program_id(0)
    x = jnp.reshape(x_ref[...], (_TILE, _CODE_DIM))
    xn = x / jnp.maximum(
        jnp.sqrt(jnp.sum(x * x, axis=1, keepdims=True)), 1e-12)
    simT = lax.dot_general(
        cbnb_ref[...], xn.astype(jnp.bfloat16),
        (((1,), (1,)), ((), ())),
        preferred_element_type=jnp.float32)
    maxv = jnp.max(simT, axis=0)
    onehot = jnp.where(simT == maxv[None, :],
                       jnp.float32(1.0), jnp.float32(0.0))
    i4 = lax.broadcasted_iota(jnp.int32, (4, _NUM_CODE), 1)
    r4 = lax.broadcasted_iota(jnp.int32, (4, _NUM_CODE), 0)
    wt = jnp.where(
        r4 == 0, i4 & 255,
        jnp.where(r4 == 1, i4 >> 8,
                  jnp.where(r4 == 2, 1, i4 * i4))).astype(jnp.float32)
    res = lax.dot_general(wt, onehot, (((1,), (0,)), ((), ())),
                          preferred_element_type=jnp.float32)
    ssum = res[0] + jnp.float32(256.0) * res[1]
    cnt = res[2]
    qsum = res[3]
    first = jnp.float32(0.5) * (
        ssum - jnp.sqrt(jnp.maximum(
            jnp.float32(2.0) * qsum - ssum * ssum, 0.0)))
    idx = jnp.where(cnt < jnp.float32(1.5), ssum, first).astype(jnp.int32)
    idx_ref[0, 0, :] = idx
    s = jnp.sum(maxv).reshape(1, 1)

    @pl.when(pid == 0)
    def _():
        maxsum_ref[...] = s

    @pl.when(pid != 0)
    def _():
        maxsum_ref[...] = maxsum_ref[...] + s


def _vq_tc(x4d, cbnb):
    b, c, t, d = x4d.shape
    grid = (b * c * t) // _TILE
    cpb = _TILE // t
    return pl.pallas_call(
        _vq_tc_body,
        grid=(grid,),
        in_specs=[
            pl.BlockSpec((1, cpb, t, d),
                         lambda i: (i // (8 // cpb), i % (8 // cpb), 0, 0)),
            pl.BlockSpec((_NUM_CODE, _CODE_DIM), lambda i: (0, 0)),
        ],
        out_specs=[
            pl.BlockSpec((1, 1, _TILE), lambda i: (i, 0, 0)),
            pl.BlockSpec((1, 1), lambda i: (0, 0)),
        ],
        out_shape=[
            jax.ShapeDtypeStruct((grid, 1, _TILE), jnp.int32),
            jax.ShapeDtypeStruct((1, 1), jnp.float32),
        ],
    )(x4d, cbnb)


def _vq_sc_body(cbn_hbm, idx_hbm, out_hbm, hist_hbm,
                idx_v, rows_v, hist16_v, hist_v, sem):
    wid = lax.axis_index("s") * _NC + lax.axis_index("c")

    pltpu.sync_copy(idx_hbm.at[wid], idx_v)

    zeros16 = jnp.zeros((_NS,), jnp.float32)

    def _zero(c, _):
        for l in range(_NS):
            hist16_v[l, pl.ds(c * _NS, _NS)] = zeros16
        return 0

    lax.fori_loop(0, _NUM_CODE // _NS, _zero, 0)

    lanes = lax.iota(jnp.int32, _NS)
    ones16 = jnp.ones((_NS,), jnp.float32)

    def _hist(g, _):
        codes = idx_v[pl.ds(g * _NS, _NS)]
        plsc.addupdate_scatter(hist16_v, [lanes, codes], ones16)
        return 0

    slots = [None] * _GROUP
    for c in range(_GROUP):
        slots[c] = pltpu.async_copy(
            cbn_hbm.at[idx_v.at[pl.ds(c * _CHUNK, _CHUNK)]],
            rows_v.at[c], sem.at[c])
    lax.fori_loop(0, _RPW // _NS, _hist, 0)
    for c in range(_NCHUNK):
        j = c % _GROUP
        slots[j].wait()
        pltpu.sync_copy(
            rows_v.at[j],
            out_hbm.at[pl.ds(wid * _RPW + c * _CHUNK, _CHUNK)])
        nxt = c + _GROUP
        if nxt < _NCHUNK:
            slots[j] = pltpu.async_copy(
                cbn_hbm.at[idx_v.at[pl.ds(nxt * _CHUNK, _CHUNK)]],
                rows_v.at[j], sem.at[j])

    def _red(c, _):
        acc = jnp.zeros((_NS,), jnp.float32)
        for l in range(_NS):
            acc = acc + hist16_v[l, pl.ds(c * _NS, _NS)]
        hist_v[0, pl.ds(c * _NS, _NS)] = acc
        return 0

    lax.fori_loop(0, _NUM_CODE // _NS, _red, 0)
    pltpu.sync_copy(hist_v, hist_hbm.at[wid].at[pl.ds(0, 1)])


def _vq_sc(cbn_pad, idx2d):
    mesh = plsc.VectorSubcoreMesh(core_axis_name="c", subcore_axis_name="s")
    f = pl.kernel(
        _vq_sc_body,
        mesh=mesh,
        compiler_params=pltpu.CompilerParams(needs_layout_passes=False),
        out_type=[
            jax.ShapeDtypeStruct((_ROWS, _PAD), jnp.float32),
            jax.ShapeDtypeStruct((_NW, 8, _NUM_CODE), jnp.float32),
        ],
        scratch_types=[
            pltpu.VMEM((_RPW,), jnp.int32),
            pltpu.VMEM((_GROUP, _CHUNK, _PAD), jnp.float32),
            pltpu.VMEM((_NS, _NUM_CODE), jnp.float32),
            pltpu.VMEM((1, _NUM_CODE), jnp.float32),
            pltpu.SemaphoreType.DMA((_GROUP,)),
        ],
    )
    return f(cbn_pad, idx2d)


def kernel(encoded_patch_input, codebook):
    b, c, t, d = encoded_patch_input.shape
    rows = b * c * t

    cbn, cbnb = _cb_prep(codebook)
    idx3, maxsum = _vq_tc(encoded_patch_input, cbnb)
    idx = idx3.reshape(rows)

    out128, hist32 = _vq_sc(cbn, idx.reshape(_NW, _RPW))
    quantized = out128[:, :_CODE_DIM]
    hist = jnp.sum(hist32[:, 0, :], axis=0)

    total = jnp.float32(rows * d)
    loss = (1.0 + _COMMIT) * (2.0 * rows - 2.0 * maxsum[0, 0]) / total
    avg_probs = hist / rows
    perplexity = jnp.exp(-jnp.sum(avg_probs * jnp.log(avg_probs + 1e-10)))
    quantized_st = quantized.reshape(b, c, t, d)
    code_indices = idx.reshape(b, c, t)
    return (loss, quantized_st, perplexity, codebook, code_indices)


# speedup vs baseline: 1.2168x; 1.0404x over previous
import jax
import jax.numpy as jnp
from jax import lax
from jax.experimental import pallas as pl
from jax.experimental.pallas import tpu as pltpu
from jax.experimental.pallas import tpu_sc as plsc

_NUM_CODE = 1024
_CODE_DIM = 64
_COMMIT = 0.25
_TILE = 2304

_NC = 2
_NS = 16
_NW = _NC * _NS
_ROWS = 73728
_RPW = _ROWS // _NW
_CHUNK = 128
_NCHUNK = _RPW // _CHUNK
_GROUP = 6
_PAD = 128


def _cb_prep_body(cb_ref, cbn_ref, cbnb_ref):
    cb = cb_ref[...]
    cbn = cb / jnp.maximum(
        jnp.sqrt(jnp.sum(cb * cb, axis=1, keepdims=True)), 1e-12)
    cbn_ref[...] = jnp.concatenate(
        [cbn, jnp.zeros((_NUM_CODE, _PAD - _CODE_DIM), jnp.float32)], axis=1)
    cbnb_ref[...] = cbn.astype(jnp.bfloat16)


def _cb_prep(codebook):
    return pl.pallas_call(
        _cb_prep_body,
        out_shape=[
            jax.ShapeDtypeStruct((_NUM_CODE, _PAD), jnp.float32),
            jax.ShapeDtypeStruct((_NUM_CODE, _CODE_DIM), jnp.bfloat16),
        ],
    )(codebook)


def _vq_tc_body(x_ref, cbnb_ref, idx_ref, maxsum_ref):
    pid = pl.program_id(0)
    x = jnp.reshape(x_ref[...], (_TILE, _CODE_DIM))
    xn = x / jnp.maximum(
        jnp.sqrt(jnp.sum(x * x, axis=1, keepdims=True)), 1e-12)
    simT = lax.dot_general(
        cbnb_ref[...], xn.astype(jnp.bfloat16),
        (((1,), (1,)), ((), ())),
        preferred_element_type=jnp.float32)
    maxv = jnp.max(simT, axis=0)
    onehot = jnp.where(simT == maxv[None, :],
                       jnp.float32(1.0), jnp.float32(0.0))
    i4 = lax.broadcasted_iota(jnp.int32, (4, _NUM_CODE), 1)
    r4 = lax.broadcasted_iota(jnp.int32, (4, _NUM_CODE), 0)
    wt = jnp.where(
        r4 == 0, i4 & 255,
        jnp.where(r4 == 1, i4 >> 8,
                  jnp.where(r4 == 2, 1, i4 * i4))).astype(jnp.float32)
    res = lax.dot_general(wt, onehot, (((1,), (0,)), ((), ())),
                          preferred_element_type=jnp.float32)
    ssum = res[0] + jnp.float32(256.0) * res[1]
    cnt = res[2]
    qsum = res[3]
    first = jnp.float32(0.5) * (
        ssum - jnp.sqrt(jnp.maximum(
            jnp.float32(2.0) * qsum - ssum * ssum, 0.0)))
    idx = jnp.where(cnt < jnp.float32(1.5), ssum, first).astype(jnp.int32)
    idx_ref[0, 0, :] = idx
    s = jnp.sum(maxv).reshape(1, 1)

    @pl.when(pid == 0)
    def _():
        maxsum_ref[...] = s

    @pl.when(pid != 0)
    def _():
        maxsum_ref[...] = maxsum_ref[...] + s


def _vq_tc(x4d, cbnb):
    b, c, t, d = x4d.shape
    grid = (b * c * t) // _TILE
    cpb = _TILE // t
    return pl.pallas_call(
        _vq_tc_body,
        grid=(grid,),
        in_specs=[
            pl.BlockSpec((1, cpb, t, d),
                         lambda i: (i // (8 // cpb), i % (8 // cpb), 0, 0)),
            pl.BlockSpec((_NUM_CODE, _CODE_DIM), lambda i: (0, 0)),
        ],
        out_specs=[
            pl.BlockSpec((1, 1, _TILE), lambda i: (i, 0, 0)),
            pl.BlockSpec((1, 1), lambda i: (0, 0)),
        ],
        out_shape=[
            jax.ShapeDtypeStruct((grid, 1, _TILE), jnp.int32),
            jax.ShapeDtypeStruct((1, 1), jnp.float32),
        ],
    )(x4d, cbnb)


def _vq_sc_body(cbn_hbm, idx_hbm, out_hbm, hist_hbm,
                idx_v, rows_v, hist16_v, hist_v, sem):
    wid = lax.axis_index("s") * _NC + lax.axis_index("c")

    pltpu.sync_copy(idx_hbm.at[wid], idx_v)

    zeros16 = jnp.zeros((_NS,), jnp.float32)

    def _zero(c, _):
        for l in range(_NS):
            hist16_v[l, pl.ds(c * _NS, _NS)] = zeros16
        return 0

    lax.fori_loop(0, _NUM_CODE // _NS, _zero, 0)

    lanes = lax.iota(jnp.int32, _NS)
    ones16 = jnp.ones((_NS,), jnp.float32)

    def _hist(g, _):
        codes = idx_v[pl.ds(g * _NS, _NS)]
        plsc.addupdate_scatter(hist16_v, [lanes, codes], ones16)
        return 0

    slots = [None] * _GROUP
    for c in range(_GROUP):
        slots[c] = pltpu.async_copy(
            cbn_hbm.at[idx_v.at[pl.ds(c * _CHUNK, _CHUNK)]],
            rows_v.at[c], sem.at[c])
    lax.fori_loop(0, _RPW // _NS, _hist, 0)
    for c in range(_NCHUNK):
        j = c % _GROUP
        slots[j].wait()
        pltpu.sync_copy(
            rows_v.at[j],
            out_hbm.at[pl.ds(wid * _RPW + c * _CHUNK, _CHUNK)])
        nxt = c + _GROUP
        if nxt < _NCHUNK:
            slots[j] = pltpu.async_copy(
                cbn_hbm.at[idx_v.at[pl.ds(nxt * _CHUNK, _CHUNK)]],
                rows_v.at[j], sem.at[j])

    def _red(c, _):
        acc = jnp.zeros((_NS,), jnp.float32)
        for l in range(_NS):
            acc = acc + hist16_v[l, pl.ds(c * _NS, _NS)]
        hist_v[0, pl.ds(c * _NS, _NS)] = acc
        return 0

    lax.fori_loop(0, _NUM_CODE // _NS, _red, 0)
    pltpu.sync_copy(hist_v, hist_hbm.at[wid].at[pl.ds(0, 1)])


def _vq_sc(cbn_pad, idx2d):
    mesh = plsc.VectorSubcoreMesh(core_axis_name="c", subcore_axis_name="s")
    f = pl.kernel(
        _vq_sc_body,
        mesh=mesh,
        compiler_params=pltpu.CompilerParams(needs_layout_passes=False),
        out_type=[
            jax.ShapeDtypeStruct((_ROWS, _PAD), jnp.float32),
            jax.ShapeDtypeStruct((_NW, 8, _NUM_CODE), jnp.float32),
        ],
        scratch_types=[
            pltpu.VMEM((_RPW,), jnp.int32),
            pltpu.VMEM((_GROUP, _CHUNK, _PAD), jnp.float32),
            pltpu.VMEM((_NS, _NUM_CODE), jnp.float32),
            pltpu.VMEM((1, _NUM_CODE), jnp.float32),
            pltpu.SemaphoreType.DMA((_GROUP,)),
        ],
    )
    return f(cbn_pad, idx2d)


def kernel(encoded_patch_input, codebook):
    b, c, t, d = encoded_patch_input.shape
    rows = b * c * t

    cbn, cbnb = _cb_prep(codebook)
    idx3, maxsum = _vq_tc(encoded_patch_input, cbnb)
    idx = idx3.reshape(rows)

    out128, hist32 = _vq_sc(cbn, idx.reshape(_NW, _RPW))
    quantized = out128[:, :_CODE_DIM]
    hist = jnp.sum(hist32[:, 0, :], axis=0)

    total = jnp.float32(rows * d)
    loss = (1.0 + _COMMIT) * (2.0 * rows - 2.0 * maxsum[0, 0]) / total
    avg_probs = hist / rows
    perplexity = jnp.exp(-jnp.sum(avg_probs * jnp.log(avg_probs + 1e-10)))
    quantized_st = quantized.reshape(b, c, t, d)
    code_indices = idx.reshape(b, c, t)
    return (loss, quantized_st, perplexity, codebook, code_indices)


# speedup vs baseline: 1.2576x; 1.0335x over previous
import jax
import jax.numpy as jnp
from jax import lax
from jax.experimental import pallas as pl
from jax.experimental.pallas import tpu as pltpu
from jax.experimental.pallas import tpu_sc as plsc

_NUM_CODE = 1024
_CODE_DIM = 64
_COMMIT = 0.25
_TILE = 4608

_NC = 2
_NS = 16
_NW = _NC * _NS
_ROWS = 73728
_RPW = _ROWS // _NW
_CHUNK = 128
_NCHUNK = _RPW // _CHUNK
_GROUP = 6
_PAD = 128


def _cb_prep_body(cb_ref, cbn_ref, cbnb_ref):
    cb = cb_ref[...]
    cbn = cb / jnp.maximum(
        jnp.sqrt(jnp.sum(cb * cb, axis=1, keepdims=True)), 1e-12)
    cbn_ref[...] = jnp.concatenate(
        [cbn, jnp.zeros((_NUM_CODE, _PAD - _CODE_DIM), jnp.float32)], axis=1)
    cbnb_ref[...] = cbn.astype(jnp.bfloat16)


def _cb_prep(codebook):
    return pl.pallas_call(
        _cb_prep_body,
        out_shape=[
            jax.ShapeDtypeStruct((_NUM_CODE, _PAD), jnp.float32),
            jax.ShapeDtypeStruct((_NUM_CODE, _CODE_DIM), jnp.bfloat16),
        ],
    )(codebook)


def _vq_tc_body(x_ref, cbnb_ref, idx_ref, maxsum_ref):
    pid = pl.program_id(0)
    x = jnp.reshape(x_ref[...], (_TILE, _CODE_DIM))
    xn = x / jnp.maximum(
        jnp.sqrt(jnp.sum(x * x, axis=1, keepdims=True)), 1e-12)
    simT = lax.dot_general(
        cbnb_ref[...], xn.astype(jnp.bfloat16),
        (((1,), (1,)), ((), ())),
        preferred_element_type=jnp.float32)
    maxv = jnp.max(simT, axis=0)
    onehot = jnp.where(simT == maxv[None, :],
                       jnp.float32(1.0), jnp.float32(0.0))
    i4 = lax.broadcasted_iota(jnp.int32, (4, _NUM_CODE), 1)
    r4 = lax.broadcasted_iota(jnp.int32, (4, _NUM_CODE), 0)
    wt = jnp.where(
        r4 == 0, i4 & 255,
        jnp.where(r4 == 1, i4 >> 8,
                  jnp.where(r4 == 2, 1, i4 * i4))).astype(jnp.float32)
    res = lax.dot_general(wt, onehot, (((1,), (0,)), ((), ())),
                          preferred_element_type=jnp.float32)
    ssum = res[0] + jnp.float32(256.0) * res[1]
    cnt = res[2]
    qsum = res[3]
    first = jnp.float32(0.5) * (
        ssum - jnp.sqrt(jnp.maximum(
            jnp.float32(2.0) * qsum - ssum * ssum, 0.0)))
    idx = jnp.where(cnt < jnp.float32(1.5), ssum, first).astype(jnp.int32)
    idx_ref[0, 0, :] = idx
    s = jnp.sum(maxv).reshape(1, 1)

    @pl.when(pid == 0)
    def _():
        maxsum_ref[...] = s

    @pl.when(pid != 0)
    def _():
        maxsum_ref[...] = maxsum_ref[...] + s


def _vq_tc(x4d, cbnb):
    b, c, t, d = x4d.shape
    grid = (b * c * t) // _TILE
    cpb = _TILE // t
    return pl.pallas_call(
        _vq_tc_body,
        grid=(grid,),
        in_specs=[
            pl.BlockSpec((1, cpb, t, d),
                         lambda i: (i // (8 // cpb), i % (8 // cpb), 0, 0)),
            pl.BlockSpec((_NUM_CODE, _CODE_DIM), lambda i: (0, 0)),
        ],
        out_specs=[
            pl.BlockSpec((1, 1, _TILE), lambda i: (i, 0, 0)),
            pl.BlockSpec((1, 1), lambda i: (0, 0)),
        ],
        out_shape=[
            jax.ShapeDtypeStruct((grid, 1, _TILE), jnp.int32),
            jax.ShapeDtypeStruct((1, 1), jnp.float32),
        ],
    )(x4d, cbnb)


def _vq_sc_body(cbn_hbm, idx_hbm, out_hbm, hist_hbm,
                idx_v, rows_v, hist16_v, hist_v, sem):
    wid = lax.axis_index("s") * _NC + lax.axis_index("c")

    pltpu.sync_copy(idx_hbm.at[wid], idx_v)

    zeros16 = jnp.zeros((_NS,), jnp.float32)

    def _zero(c, _):
        for l in range(_NS):
            hist16_v[l, pl.ds(c * _NS, _NS)] = zeros16
        return 0

    lax.fori_loop(0, _NUM_CODE // _NS, _zero, 0)

    lanes = lax.iota(jnp.int32, _NS)
    ones16 = jnp.ones((_NS,), jnp.float32)

    def _hist(g, _):
        codes = idx_v[pl.ds(g * _NS, _NS)]
        plsc.addupdate_scatter(hist16_v, [lanes, codes], ones16)
        return 0

    slots = [None] * _GROUP
    for c in range(_GROUP):
        slots[c] = pltpu.async_copy(
            cbn_hbm.at[idx_v.at[pl.ds(c * _CHUNK, _CHUNK)]],
            rows_v.at[c], sem.at[c])
    lax.fori_loop(0, _RPW // _NS, _hist, 0)
    for c in range(_NCHUNK):
        j = c % _GROUP
        slots[j].wait()
        pltpu.sync_copy(
            rows_v.at[j],
            out_hbm.at[pl.ds(wid * _RPW + c * _CHUNK, _CHUNK)])
        nxt = c + _GROUP
        if nxt < _NCHUNK:
            slots[j] = pltpu.async_copy(
                cbn_hbm.at[idx_v.at[pl.ds(nxt * _CHUNK, _CHUNK)]],
                rows_v.at[j], sem.at[j])

    def _red(c, _):
        acc = jnp.zeros((_NS,), jnp.float32)
        for l in range(_NS):
            acc = acc + hist16_v[l, pl.ds(c * _NS, _NS)]
        hist_v[0, pl.ds(c * _NS, _NS)] = acc
        return 0

    lax.fori_loop(0, _NUM_CODE // _NS, _red, 0)
    pltpu.sync_copy(hist_v, hist_hbm.at[wid].at[pl.ds(0, 1)])


def _vq_sc(cbn_pad, idx2d):
    mesh = plsc.VectorSubcoreMesh(core_axis_name="c", subcore_axis_name="s")
    f = pl.kernel(
        _vq_sc_body,
        mesh=mesh,
        compiler_params=pltpu.CompilerParams(needs_layout_passes=False),
        out_type=[
            jax.ShapeDtypeStruct((_ROWS, _PAD), jnp.float32),
            jax.ShapeDtypeStruct((_NW, 8, _NUM_CODE), jnp.float32),
        ],
        scratch_types=[
            pltpu.VMEM((_RPW,), jnp.int32),
            pltpu.VMEM((_GROUP, _CHUNK, _PAD), jnp.float32),
            pltpu.VMEM((_NS, _NUM_CODE), jnp.float32),
            pltpu.VMEM((1, _NUM_CODE), jnp.float32),
            pltpu.SemaphoreType.DMA((_GROUP,)),
        ],
    )
    return f(cbn_pad, idx2d)


def kernel(encoded_patch_input, codebook):
    b, c, t, d = encoded_patch_input.shape
    rows = b * c * t

    cbn, cbnb = _cb_prep(codebook)
    idx3, maxsum = _vq_tc(encoded_patch_input, cbnb)
    idx = idx3.reshape(rows)

    out128, hist32 = _vq_sc(cbn, idx.reshape(_NW, _RPW))
    quantized = out128[:, :_CODE_DIM]
    hist = jnp.sum(hist32[:, 0, :], axis=0)

    total = jnp.float32(rows * d)
    loss = (1.0 + _COMMIT) * (2.0 * rows - 2.0 * maxsum[0, 0]) / total
    avg_probs = hist / rows
    perplexity = jnp.exp(-jnp.sum(avg_probs * jnp.log(avg_probs + 1e-10)))
    quantized_st = quantized.reshape(b, c, t, d)
    code_indices = idx.reshape(b, c, t)
    return (loss, quantized_st, perplexity, codebook, code_indices)
